# baseline (speedup 1.0000x reference)
import functools

import jax
import jax.numpy as jnp
from jax import lax
from jax.experimental import pallas as pl
from jax.experimental.pallas import tpu as pltpu
from jax.experimental.pallas import tpu_sc as plsc

NC = 2
NS = 16
NW = NC * NS
LANES = 16

SC_PLANES = 256


def _make_sc_shuffle(n_planes, P, C):
    planes_per_w = n_planes // NW
    n_groups = P // LANES
    TC_ = C // 128
    mesh = plsc.VectorSubcoreMesh(core_axis_name="c", subcore_axis_name="s")

    @functools.partial(
        pl.kernel,
        out_type=jax.ShapeDtypeStruct((n_planes * P,), jnp.float32),
        mesh=mesh,
        compiler_params=pltpu.CompilerParams(needs_layout_passes=False),
        scratch_types=[
            pltpu.VMEM((C,), jnp.int32),
            pltpu.VMEM((P,), jnp.int32),
            pltpu.VMEM((P,), jnp.float32),
            pltpu.VMEM((P,), jnp.float32),
            pltpu.VMEM((P,), jnp.float32),
            pltpu.VMEM((P,), jnp.float32),
            pltpu.SemaphoreType.DMA,
            pltpu.SemaphoreType.DMA,
            pltpu.SemaphoreType.DMA,
            pltpu.SemaphoreType.DMA,
        ],
    )
    def sc_shuffle(x_hbm, idx_hbm, out_hbm, idx_v, gidx_v,
                   in0, in1, out0, out1, si0, si1, so0, so1):
        wid = lax.axis_index("s") * NC + lax.axis_index("c")
        base = wid * planes_per_w
        pltpu.sync_copy(idx_hbm, idx_v)
        iota = lax.iota(jnp.int32, LANES)

        def build(j, _):
            p = j * LANES + iota
            t = p >> 10
            twtc = lax.div(t, TC_) * TC_
            c_out = (t - twtc) * 128 + (p & 127)
            c_src = plsc.load_gather(idx_v, [c_out])
            gidx_v[pl.ds(j * LANES, LANES)] = (
                p
                + (c_src + ((c_src >> 7) * 896))
                - (c_out + ((c_out >> 7) * 896))
            )
            return 0

        lax.fori_loop(0, n_groups, build, 0, unroll=4)

        def start_in(i, buf, sem):
            pltpu.make_async_copy(
                x_hbm.at[pl.ds((base + i) * P, P)], buf, sem).start()

        def wait_in(i, buf, sem):
            pltpu.make_async_copy(
                x_hbm.at[pl.ds((base + i) * P, P)], buf, sem).wait()

        def start_out(i, buf, sem):
            pltpu.make_async_copy(
                buf, out_hbm.at[pl.ds((base + i) * P, P)], sem).start()

        def wait_out(i, buf, sem):
            pltpu.make_async_copy(
                buf, out_hbm.at[pl.ds((base + i) * P, P)], sem).wait()

        def permute(src, dst):
            def step(j, _):
                o = j * LANES
                dst[pl.ds(o, LANES)] = plsc.load_gather(
                    src, [gidx_v[pl.ds(o, LANES)]])
                return 0

            lax.fori_loop(0, n_groups, step, 0, unroll=8)

        start_in(0, in0, si0)
        start_in(1, in1, si1)
        wait_in(0, in0, si0)
        permute(in0, out0)
        start_out(0, out0, so0)
        start_in(2, in0, si0)
        wait_in(1, in1, si1)
        permute(in1, out1)
        start_out(1, out1, so1)
        start_in(3, in1, si1)

        def pair(k, _):
            i0 = 2 * k
            wait_in(i0, in0, si0)
            wait_out(i0 - 2, out0, so0)
            permute(in0, out0)
            start_out(i0, out0, so0)
            start_in(i0 + 2, in0, si0)
            wait_in(i0 + 1, in1, si1)
            wait_out(i0 - 1, out1, so1)
            permute(in1, out1)
            start_out(i0 + 1, out1, so1)
            start_in(i0 + 3, in1, si1)
            return 0

        lax.fori_loop(1, planes_per_w // 2 - 1, pair, 0)

        i0 = planes_per_w - 2
        wait_in(i0, in0, si0)
        wait_out(i0 - 2, out0, so0)
        permute(in0, out0)
        start_out(i0, out0, so0)
        wait_in(i0 + 1, in1, si1)
        wait_out(i0 - 1, out1, so1)
        permute(in1, out1)
        start_out(i0 + 1, out1, so1)
        wait_out(i0, out0, so0)
        wait_out(i0 + 1, out1, so1)

    return sc_shuffle


def _onehot_body(idx_ref, p_ref):
    r = lax.broadcasted_iota(jnp.int32, p_ref.shape, 0)
    idx_row = idx_ref[...].reshape(1, -1)
    p_ref[...] = (r == idx_row).astype(jnp.float32)


def _mm_body(x_ref, p_ref, o_ref):
    o_ref[...] = lax.dot_general(
        x_ref[...], p_ref[...], (((1,), (0,)), ((), ())),
        preferred_element_type=jnp.float32,
    )


def kernel(x, forward_shuffle_idx):
    B, C, H, W = x.shape
    TW, TC_ = W // 8, C // 128
    P = TW * TC_ * 1024
    n_planes = B * H
    M_sc = SC_PLANES * W
    M_tc = (n_planes - SC_PLANES) * W

    xm = x.transpose(0, 2, 3, 1).reshape(B * H * W, C)

    xp = (
        x.transpose(0, 2, 3, 1)
        .reshape(B, H, TW, 8, TC_, 128)
        .transpose(0, 1, 2, 4, 3, 5)
        .reshape(-1)
    )
    sc_out = _make_sc_shuffle(SC_PLANES, P, C)(
        xp[: SC_PLANES * P], forward_shuffle_idx)

    p_mat = pl.pallas_call(
        _onehot_body,
        out_shape=jax.ShapeDtypeStruct((C, C), jnp.float32),
    )(forward_shuffle_idx.reshape(1, C))

    BM = 2048
    OFF = M_sc // BM
    tc_out = pl.pallas_call(
        _mm_body,
        grid=(M_tc // BM,),
        in_specs=[
            pl.BlockSpec((BM, C), lambda m: (m + OFF, 0)),
            pl.BlockSpec((C, C), lambda m: (0, 0)),
        ],
        out_specs=pl.BlockSpec((BM, C), lambda m: (m, 0)),
        out_shape=jax.ShapeDtypeStruct((M_tc, C), jnp.float32),
    )(xm, p_mat)

    sc_part = (
        sc_out.reshape(SC_PLANES, TW, TC_, 8, 128)
        .transpose(0, 1, 3, 2, 4)
        .reshape(M_sc, C)
    )
    out = jnp.concatenate([sc_part, tc_out], axis=0)
    out = out.reshape(B, H, W, C).transpose(0, 3, 1, 2)
    return (out, jnp.zeros((), x.dtype))


# speedup vs baseline: 2.3895x; 2.3895x over previous
import jax
import jax.numpy as jnp
from jax import lax
from jax.experimental import pallas as pl


def _onehot_body(idx_ref, p_ref):
    r = lax.broadcasted_iota(jnp.int32, p_ref.shape, 0)
    idx_row = idx_ref[...].reshape(1, -1)
    p_ref[...] = (r == idx_row).astype(jnp.float32)


def _mm_body(x_ref, p_ref, o_ref):
    o_ref[...] = lax.dot_general(
        x_ref[...], p_ref[...], (((1,), (0,)), ((), ())),
        preferred_element_type=jnp.float32,
    )


def kernel(x, forward_shuffle_idx):
    B, C, H, W = x.shape
    M = B * H * W
    xm = x.transpose(0, 2, 3, 1).reshape(M, C)

    p_mat = pl.pallas_call(
        _onehot_body,
        out_shape=jax.ShapeDtypeStruct((C, C), jnp.float32),
    )(forward_shuffle_idx.reshape(1, C))

    BM = 2048
    out = pl.pallas_call(
        _mm_body,
        grid=(M // BM,),
        in_specs=[
            pl.BlockSpec((BM, C), lambda m: (m, 0)),
            pl.BlockSpec((C, C), lambda m: (0, 0)),
        ],
        out_specs=pl.BlockSpec((BM, C), lambda m: (m, 0)),
        out_shape=jax.ShapeDtypeStruct((M, C), jnp.float32),
    )(xm, p_mat)

    out = out.reshape(B, H, W, C).transpose(0, 3, 1, 2)
    return (out, jnp.zeros((), x.dtype))
